# baseline (speedup 1.0000x reference)
import functools

import jax
import jax.numpy as jnp
from jax import lax
from jax.experimental import pallas as pl
from jax.experimental.pallas import tpu as pltpu
from jax.experimental.pallas import tpu_sc as plsc

NSEG = 16
N_TOK = 32768
D = 128
L = 16
KD = D // L
NC = 2
NS = 16
NW = NC * NS
ROWS_PER_W = N_TOK // NW
CHUNK = 256
NCHUNK = ROWS_PER_W // CHUNK
VPC = CHUNK // L

NEG_INF = float("-inf")


def _partials_body(x_hbm, b_hbm, w_hbm, a_out, s_out,
                   xb, bb, wb, sb, mref, dref, cref, scaleref,
                   accr, srow, sem0, sem1):
  cid = lax.axis_index("c")
  sid = lax.axis_index("s")
  wid = sid * NC + cid
  base = wid * ROWS_PER_W

  pltpu.sync_copy(b_hbm.at[pl.ds(base, ROWS_PER_W)], bb)
  pltpu.sync_copy(w_hbm, wb)

  def _init(i, _):
    for k in range(KD):
      accr[i, pl.ds(k * L, L)] = jnp.zeros((L,), jnp.float32)
    return 0
  lax.fori_loop(0, NSEG, _init, 0)
  mref[...] = jnp.full((L,), NEG_INF, jnp.float32)
  dref[...] = jnp.zeros((L,), jnp.float32)
  cref[...] = jnp.zeros((L,), jnp.float32)

  wv = [wb[pl.ds(k * L, L)] for k in range(KD)]
  lane = lax.iota(jnp.int32, L)
  zidx = jnp.zeros((L,), jnp.int32)

  sems = [sem0, sem1]

  def _copy(c):
    buf = c % 2
    return pltpu.make_async_copy(
        x_hbm.at[pl.ds(base + c * CHUNK, CHUNK)], xb.at[buf], sems[buf])

  _copy(0).start()

  for c in range(NCHUNK):
    buf = c % 2
    _copy(c).wait()
    if c + 1 < NCHUNK:
      _copy(c + 1).start()

    with jax.named_scope("score"):
      @plsc.parallel_loop(0, VPC, unroll=2)
      def _score(i):
        svec = jnp.zeros((L,), jnp.float32)
        for j in range(L):
          r = i * L + j
          acc = xb[buf, r, pl.ds(0, L)] * wv[0]
          for k in range(1, KD):
            acc = acc + xb[buf, r, pl.ds(k * L, L)] * wv[k]
          svec = jnp.where(lane == j, jnp.sum(acc), svec)
        sb[pl.ds(i * L, L)] = svec

    with jax.named_scope("stats"):
      b0 = bb[pl.ds(c * CHUNK, L)][0]
      b1 = bb[pl.ds(c * CHUNK + CHUNK - L, L)][L - 1]

      def _segmax(seg, mchunk):
        def vb(i, acc):
          bvec = bb[pl.ds(c * CHUNK + i * L, L)]
          svec = sb[pl.ds(i * L, L)]
          return jnp.maximum(acc, jnp.where(bvec == seg, svec, NEG_INF))
        acc = lax.fori_loop(0, VPC, vb, jnp.full((L,), NEG_INF, jnp.float32))
        return jnp.where(lane == seg, jnp.max(acc), mchunk)
      m_chunk = lax.fori_loop(b0, b1 + 1, _segmax,
                              jnp.full((L,), NEG_INF, jnp.float32))

      m_old = mref[...]
      m_new = jnp.maximum(m_old, m_chunk)
      scale = jnp.where(m_old == NEG_INF, 0.0, jnp.exp(m_old - m_new))
      mref[...] = m_new
      scaleref[...] = scale
      dref[...] = dref[...] * scale

      def _rescale(seg, _):
        sc = plsc.load_gather(scaleref, [jnp.zeros((L,), jnp.int32) + seg])
        for k in range(KD):
          accr[seg, pl.ds(k * L, L)] = accr[seg, pl.ds(k * L, L)] * sc
        return 0
      lax.fori_loop(b0, b1 + 1, _rescale, 0)

    def _seg_run(seg, run_start):
      def cb(i, acc):
        bvec = bb[pl.ds(c * CHUNK + i * L, L)]
        return acc + plsc.all_reduce_population_count(bvec == seg)
      cnt = lax.fori_loop(0, VPC, cb, jnp.zeros((L,), jnp.int32))
      nseg = cnt[0]
      m_b = plsc.load_gather(mref, [zidx + seg])

      def rb(r, carry):
        accs, dacc = carry
        eb = jnp.exp(plsc.load_gather(sb, [zidx + r]) - m_b)
        new = tuple(accs[k] + eb * xb[buf, r, pl.ds(k * L, L)]
                    for k in range(KD))
        return new, dacc + eb[0]
      init = (tuple(jnp.zeros((L,), jnp.float32) for _ in range(KD)),
              jnp.zeros((), jnp.float32))
      accs, dacc = lax.fori_loop(run_start, run_start + nseg, rb, init)
      for k in range(KD):
        plsc.addupdate(accr.at[seg, pl.ds(k * L, L)], accs[k])
      dref[...] = dref[...] + jnp.where(lane == seg, dacc, 0.0)
      cf = nseg.astype(jnp.float32)
      cref[...] = cref[...] + jnp.where(lane == seg, cf, 0.0)
      return run_start + nseg

    with jax.named_scope("runs"):
      lax.fori_loop(b0, b1 + 1, _seg_run, jnp.zeros((), jnp.int32))

  srow[0, :] = mref[...]
  srow[1, :] = dref[...]
  srow[2, :] = cref[...]
  pltpu.sync_copy(accr, a_out.at[wid])
  pltpu.sync_copy(srow, s_out.at[wid])


@functools.partial(jax.jit, static_argnums=())
def _partials(x, batch, w):
  mesh = plsc.VectorSubcoreMesh(core_axis_name="c", subcore_axis_name="s")
  f = pl.kernel(
      _partials_body,
      out_type=(
          jax.ShapeDtypeStruct((NW, NSEG, D), jnp.float32),
          jax.ShapeDtypeStruct((NW, 3, L), jnp.float32),
      ),
      mesh=mesh,
      scratch_types=[
          pltpu.VMEM((2, CHUNK, D), jnp.float32),
          pltpu.VMEM((ROWS_PER_W,), jnp.int32),
          pltpu.VMEM((D,), jnp.float32),
          pltpu.VMEM((CHUNK,), jnp.float32),
          pltpu.VMEM((L,), jnp.float32),
          pltpu.VMEM((L,), jnp.float32),
          pltpu.VMEM((L,), jnp.float32),
          pltpu.VMEM((L,), jnp.float32),
          pltpu.VMEM((NSEG, D), jnp.float32),
          pltpu.VMEM((3, L), jnp.float32),
          pltpu.SemaphoreType.DMA,
          pltpu.SemaphoreType.DMA,
      ],
      compiler_params=pltpu.CompilerParams(needs_layout_passes=False),
  )
  return f(x, batch, w)


def _combine_body(a_ref, s_ref, o_ref):
  m_w = s_ref[:, 0, :]
  d_w = s_ref[:, 1, :]
  c_w = s_ref[:, 2, :]
  m_g = jnp.max(m_w, axis=0)
  scale = jnp.exp(m_w - m_g[None, :])
  d_g = jnp.sum(d_w * scale, axis=0)
  c_g = jnp.sum(c_w, axis=0)
  acc = jnp.sum(a_ref[...] * scale[:, :, None], axis=0)
  o_ref[...] = acc / (d_g * c_g)[:, None]


def _combine(a, s):
  return pl.pallas_call(
      _combine_body,
      out_shape=jax.ShapeDtypeStruct((NSEG, D), jnp.float32),
  )(a, s)


def kernel(x, batch, att_w):
  w = att_w.reshape(D)
  a, s = _partials(x, batch, w)
  g = _combine(a, s)
  return (g, att_w)


# speedup vs baseline: 1.0285x; 1.0285x over previous
import functools

import jax
import jax.numpy as jnp
from jax import lax
from jax.experimental import pallas as pl
from jax.experimental.pallas import tpu as pltpu
from jax.experimental.pallas import tpu_sc as plsc

NSEG = 16
N_TOK = 32768
D = 128
L = 16
KD = D // L
NC = 2
NS = 16
NW = NC * NS
ROWS_PER_W = N_TOK // NW
CHUNK = 256
NCHUNK = ROWS_PER_W // CHUNK
VPC = CHUNK // L

NEG_INF = float("-inf")


def _partials_body(x_hbm, b_hbm, w_hbm, a_out, s_out,
                   xb, bb, wb, sb, mref, dref, cref, scaleref,
                   accr, srow, sem0, sem1):
  cid = lax.axis_index("c")
  sid = lax.axis_index("s")
  wid = sid * NC + cid
  base = wid * ROWS_PER_W

  pltpu.sync_copy(b_hbm.at[pl.ds(base, ROWS_PER_W)], bb)
  pltpu.sync_copy(w_hbm.at[0], wb)

  def _init(i, _):
    for k in range(KD):
      accr[i, pl.ds(k * L, L)] = jnp.zeros((L,), jnp.float32)
    return 0
  lax.fori_loop(0, NSEG, _init, 0)
  mref[...] = jnp.full((L,), NEG_INF, jnp.float32)
  dref[...] = jnp.zeros((L,), jnp.float32)
  cref[...] = jnp.zeros((L,), jnp.float32)

  wv = [wb[pl.ds(k * L, L)] for k in range(KD)]
  lane = lax.iota(jnp.int32, L)
  zidx = jnp.zeros((L,), jnp.int32)

  sems = [sem0, sem1]

  def _copy(c):
    buf = c % 2
    return pltpu.make_async_copy(
        x_hbm.at[pl.ds(base + c * CHUNK, CHUNK)], xb.at[buf], sems[buf])

  _copy(0).start()

  for c in range(NCHUNK):
    buf = c % 2
    _copy(c).wait()
    if c + 1 < NCHUNK:
      _copy(c + 1).start()

    def _score(i, _):
      svec = jnp.zeros((L,), jnp.float32)
      for j in range(L):
        r = i * L + j
        acc = xb[buf, r, pl.ds(0, L)] * wv[0]
        for k in range(1, KD):
          acc = acc + xb[buf, r, pl.ds(k * L, L)] * wv[k]
        svec = jnp.where(lane == j, jnp.sum(acc), svec)
      sb[pl.ds(i * L, L)] = svec
      return 0
    lax.fori_loop(0, VPC, _score, 0)

    b0 = bb[pl.ds(c * CHUNK, L)][0]
    b1 = bb[pl.ds(c * CHUNK + CHUNK - L, L)][L - 1]

    def _segmax(seg, mchunk):
      def vb(i, acc):
        bvec = bb[pl.ds(c * CHUNK + i * L, L)]
        svec = sb[pl.ds(i * L, L)]
        return jnp.maximum(acc, jnp.where(bvec == seg, svec, NEG_INF))
      acc = lax.fori_loop(0, VPC, vb, jnp.full((L,), NEG_INF, jnp.float32))
      return jnp.where(lane == seg, jnp.max(acc), mchunk)
    m_chunk = lax.fori_loop(b0, b1 + 1, _segmax,
                            jnp.full((L,), NEG_INF, jnp.float32))

    m_old = mref[...]
    m_new = jnp.maximum(m_old, m_chunk)
    scale = jnp.where(m_old == NEG_INF, 0.0, jnp.exp(m_old - m_new))
    mref[...] = m_new
    scaleref[...] = scale
    dref[...] = dref[...] * scale

    def _rescale(seg, _):
      sc = plsc.load_gather(scaleref, [jnp.zeros((L,), jnp.int32) + seg])
      for k in range(KD):
        accr[seg, pl.ds(k * L, L)] = accr[seg, pl.ds(k * L, L)] * sc
      return 0
    lax.fori_loop(b0, b1 + 1, _rescale, 0)

    def _seg_run(seg, run_start):
      def cb(i, acc):
        bvec = bb[pl.ds(c * CHUNK + i * L, L)]
        return acc + plsc.all_reduce_population_count(bvec == seg)
      cnt = lax.fori_loop(0, VPC, cb, jnp.zeros((L,), jnp.int32))
      nseg = cnt[0]
      m_b = plsc.load_gather(mref, [zidx + seg])

      def rb(r, carry):
        accs, dacc = carry
        eb = jnp.exp(plsc.load_gather(sb, [zidx + r]) - m_b)
        new = tuple(accs[k] + eb * xb[buf, r, pl.ds(k * L, L)]
                    for k in range(KD))
        return new, dacc + eb
      init = (tuple(jnp.zeros((L,), jnp.float32) for _ in range(KD)),
              jnp.zeros((L,), jnp.float32))
      accs, dacc = lax.fori_loop(run_start, run_start + nseg, rb, init)
      for k in range(KD):
        plsc.addupdate(accr.at[seg, pl.ds(k * L, L)], accs[k])
      dref[...] = dref[...] + jnp.where(lane == seg, dacc, 0.0)
      cf = nseg.astype(jnp.float32)
      cref[...] = cref[...] + jnp.where(lane == seg, cf, 0.0)
      return run_start + nseg

    lax.fori_loop(b0, b1 + 1, _seg_run, jnp.zeros((), jnp.int32))

  srow[0, :] = mref[...]
  srow[1, :] = dref[...]
  srow[2, :] = cref[...]
  pltpu.sync_copy(accr, a_out.at[wid])
  pltpu.sync_copy(srow, s_out.at[wid])


@functools.partial(jax.jit, static_argnums=())
def _partials(x, batch, w):
  mesh = plsc.VectorSubcoreMesh(core_axis_name="c", subcore_axis_name="s")
  f = pl.kernel(
      _partials_body,
      out_type=(
          jax.ShapeDtypeStruct((NW, NSEG, D), jnp.float32),
          jax.ShapeDtypeStruct((NW, 3, L), jnp.float32),
      ),
      mesh=mesh,
      scratch_types=[
          pltpu.VMEM((2, CHUNK, D), jnp.float32),
          pltpu.VMEM((ROWS_PER_W,), jnp.int32),
          pltpu.VMEM((D,), jnp.float32),
          pltpu.VMEM((CHUNK,), jnp.float32),
          pltpu.VMEM((L,), jnp.float32),
          pltpu.VMEM((L,), jnp.float32),
          pltpu.VMEM((L,), jnp.float32),
          pltpu.VMEM((L,), jnp.float32),
          pltpu.VMEM((NSEG, D), jnp.float32),
          pltpu.VMEM((3, L), jnp.float32),
          pltpu.SemaphoreType.DMA,
          pltpu.SemaphoreType.DMA,
      ],
      compiler_params=pltpu.CompilerParams(needs_layout_passes=False),
  )
  return f(x, batch, w)


def _combine_body(a_ref, s_ref, o_ref):
  m_w = s_ref[:, 0, :]
  d_w = s_ref[:, 1, :]
  c_w = s_ref[:, 2, :]
  m_g = jnp.max(m_w, axis=0)
  scale = jnp.exp(m_w - m_g[None, :])
  d_g = jnp.sum(d_w * scale, axis=0)
  c_g = jnp.sum(c_w, axis=0)
  acc = jnp.sum(a_ref[...] * scale[:, :, None], axis=0)
  o_ref[...] = acc / (d_g * c_g)[:, None]


def _combine(a, s):
  return pl.pallas_call(
      _combine_body,
      out_shape=jax.ShapeDtypeStruct((NSEG, D), jnp.float32),
  )(a, s)


def kernel(x, batch, att_w):
  a, s = _partials(x, batch, att_w)
  g = _combine(a, s)
  return (g, att_w)


# speedup vs baseline: 1.0418x; 1.0130x over previous
import functools

import jax
import jax.numpy as jnp
from jax import lax
from jax.experimental import pallas as pl
from jax.experimental.pallas import tpu as pltpu
from jax.experimental.pallas import tpu_sc as plsc

NSEG = 16
N_TOK = 32768
D = 128
L = 16
KD = D // L
NC = 2
NS = 16
NW = NC * NS
ROWS_PER_W = N_TOK // NW
CHUNK = 256
NCHUNK = ROWS_PER_W // CHUNK
VPC = CHUNK // L

NEG_INF = float("-inf")


def _partials_body(x_hbm, b_hbm, w_hbm, a_out, s_out,
                   xb, bb, wb, sb, mref, dref, cref, scaleref,
                   accr, srow, sem0, sem1):
  cid = lax.axis_index("c")
  sid = lax.axis_index("s")
  wid = sid * NC + cid
  base = wid * ROWS_PER_W

  pltpu.sync_copy(b_hbm.at[pl.ds(base, ROWS_PER_W)], bb)
  pltpu.sync_copy(w_hbm.at[0], wb)

  def _init(i, _):
    for k in range(KD):
      accr[i, pl.ds(k * L, L)] = jnp.zeros((L,), jnp.float32)
    return 0
  lax.fori_loop(0, NSEG, _init, 0)
  mref[...] = jnp.full((L,), NEG_INF, jnp.float32)
  dref[...] = jnp.zeros((L,), jnp.float32)
  cref[...] = jnp.zeros((L,), jnp.float32)

  wv = [wb[pl.ds(k * L, L)] for k in range(KD)]
  lane = lax.iota(jnp.int32, L)
  zidx = jnp.zeros((L,), jnp.int32)

  sems = [sem0, sem1]

  def _copy(c):
    buf = c % 2
    return pltpu.make_async_copy(
        x_hbm.at[pl.ds(base + c * CHUNK, CHUNK)], xb.at[buf], sems[buf])

  _copy(0).start()

  for c in range(NCHUNK):
    buf = c % 2
    _copy(c).wait()
    if c + 1 < NCHUNK:
      _copy(c + 1).start()

    def _score(i, _):
      svec = jnp.zeros((L,), jnp.float32)
      for j in range(L):
        r = i * L + j
        p = [xb[buf, r, pl.ds(k * L, L)] * wv[k] for k in range(KD)]
        while len(p) > 1:
          p = [p[t] + p[t + 1] for t in range(0, len(p), 2)]
        svec = jnp.where(lane == j, jnp.sum(p[0]), svec)
      sb[pl.ds(i * L, L)] = svec
      return 0
    lax.fori_loop(0, VPC, _score, 0)

    b0 = bb[pl.ds(c * CHUNK, L)][0]
    b1 = bb[pl.ds(c * CHUNK + CHUNK - L, L)][L - 1]

    def _segmax(seg, mchunk):
      def vb(i, acc):
        bvec = bb[pl.ds(c * CHUNK + i * L, L)]
        svec = sb[pl.ds(i * L, L)]
        return jnp.maximum(acc, jnp.where(bvec == seg, svec, NEG_INF))
      acc = lax.fori_loop(0, VPC, vb, jnp.full((L,), NEG_INF, jnp.float32))
      return jnp.where(lane == seg, jnp.max(acc), mchunk)
    m_chunk = lax.fori_loop(b0, b1 + 1, _segmax,
                            jnp.full((L,), NEG_INF, jnp.float32))

    m_old = mref[...]
    m_new = jnp.maximum(m_old, m_chunk)
    scale = jnp.where(m_old == NEG_INF, 0.0, jnp.exp(m_old - m_new))
    mref[...] = m_new
    scaleref[...] = scale
    dref[...] = dref[...] * scale

    def _rescale(seg, _):
      sc = plsc.load_gather(scaleref, [jnp.zeros((L,), jnp.int32) + seg])
      for k in range(KD):
        accr[seg, pl.ds(k * L, L)] = accr[seg, pl.ds(k * L, L)] * sc
      return 0
    lax.fori_loop(b0, b1 + 1, _rescale, 0)

    def _seg_run(seg, run_start):
      def cb(i, acc):
        bvec = bb[pl.ds(c * CHUNK + i * L, L)]
        return acc + plsc.all_reduce_population_count(bvec == seg)
      cnt = lax.fori_loop(0, VPC, cb, jnp.zeros((L,), jnp.int32))
      nseg = cnt[0]
      m_b = plsc.load_gather(mref, [zidx + seg])

      def rb(r, carry):
        accs, dacc = carry
        eb = jnp.exp(plsc.load_gather(sb, [zidx + r]) - m_b)
        new = tuple(accs[k] + eb * xb[buf, r, pl.ds(k * L, L)]
                    for k in range(KD))
        return new, dacc + eb
      init = (tuple(jnp.zeros((L,), jnp.float32) for _ in range(KD)),
              jnp.zeros((L,), jnp.float32))
      accs, dacc = lax.fori_loop(run_start, run_start + nseg, rb, init)
      for k in range(KD):
        plsc.addupdate(accr.at[seg, pl.ds(k * L, L)], accs[k])
      dref[...] = dref[...] + jnp.where(lane == seg, dacc, 0.0)
      cf = nseg.astype(jnp.float32)
      cref[...] = cref[...] + jnp.where(lane == seg, cf, 0.0)
      return run_start + nseg

    lax.fori_loop(b0, b1 + 1, _seg_run, jnp.zeros((), jnp.int32))

  srow[0, :] = mref[...]
  srow[1, :] = dref[...]
  srow[2, :] = cref[...]
  pltpu.sync_copy(accr, a_out.at[wid])
  pltpu.sync_copy(srow, s_out.at[wid])


@functools.partial(jax.jit, static_argnums=())
def _partials(x, batch, w):
  mesh = plsc.VectorSubcoreMesh(core_axis_name="c", subcore_axis_name="s")
  f = pl.kernel(
      _partials_body,
      out_type=(
          jax.ShapeDtypeStruct((NW, NSEG, D), jnp.float32),
          jax.ShapeDtypeStruct((NW, 3, L), jnp.float32),
      ),
      mesh=mesh,
      scratch_types=[
          pltpu.VMEM((2, CHUNK, D), jnp.float32),
          pltpu.VMEM((ROWS_PER_W,), jnp.int32),
          pltpu.VMEM((D,), jnp.float32),
          pltpu.VMEM((CHUNK,), jnp.float32),
          pltpu.VMEM((L,), jnp.float32),
          pltpu.VMEM((L,), jnp.float32),
          pltpu.VMEM((L,), jnp.float32),
          pltpu.VMEM((L,), jnp.float32),
          pltpu.VMEM((NSEG, D), jnp.float32),
          pltpu.VMEM((3, L), jnp.float32),
          pltpu.SemaphoreType.DMA,
          pltpu.SemaphoreType.DMA,
      ],
      compiler_params=pltpu.CompilerParams(needs_layout_passes=False),
  )
  return f(x, batch, w)


def _combine_body(a_ref, s_ref, o_ref):
  m_w = s_ref[:, 0, :]
  d_w = s_ref[:, 1, :]
  c_w = s_ref[:, 2, :]
  m_g = jnp.max(m_w, axis=0)
  scale = jnp.exp(m_w - m_g[None, :])
  d_g = jnp.sum(d_w * scale, axis=0)
  c_g = jnp.sum(c_w, axis=0)
  acc = jnp.sum(a_ref[...] * scale[:, :, None], axis=0)
  o_ref[...] = acc / (d_g * c_g)[:, None]


def _combine(a, s):
  return pl.pallas_call(
      _combine_body,
      out_shape=jax.ShapeDtypeStruct((NSEG, D), jnp.float32),
  )(a, s)


def kernel(x, batch, att_w):
  a, s = _partials(x, batch, att_w)
  g = _combine(a, s)
  return (g, att_w)
